# Initial kernel scaffold; baseline (speedup 1.0000x reference)
import functools

import jax
import jax.numpy as jnp
from jax import lax
from jax.experimental import pallas as pl
from jax.experimental.pallas import tpu as pltpu
from jax.experimental.pallas import tpu_sc as plsc

N_NODES = 10000
N_EDGES = 320000
DIM = 128

N_PAD = 10240
BLK = 2048
ACC_W = 144

NW = 32
EPW = N_EDGES // NW
C = 80
NCHUNK = EPW // C
ROWS_PER_SUB = N_NODES // 16
ZROWS = 125


def _prep_body(x_ref, w_ref, ab_ref, h_ref, hsd_ref, ms_ref, md_ref):
    i = pl.program_id(0)
    h = jnp.dot(x_ref[...], w_ref[...], preferred_element_type=jnp.float32)
    h_ref[...] = h
    hs = jnp.sum(h * ab_ref[0, :][None, :], axis=1)
    hd = jnp.sum(h * ab_ref[1, :][None, :], axis=1)
    hsd_ref[0] = hs.reshape(BLK // 128, 128)
    hsd_ref[1] = hd.reshape(BLK // 128, 128)

    @pl.when(i == 0)
    def _():
        ms_ref[...] = jnp.full((8, 128), -1e30, jnp.float32)
        md_ref[...] = jnp.full((8, 128), -1e30, jnp.float32)

    ms_ref[...] = jnp.maximum(ms_ref[...], jnp.max(hs))
    md_ref[...] = jnp.maximum(md_ref[...], jnp.max(hd))


def _prep(x_pad, W, ab):
    return pl.pallas_call(
        _prep_body,
        grid=(N_PAD // BLK,),
        in_specs=[
            pl.BlockSpec((BLK, DIM), lambda i: (i, 0)),
            pl.BlockSpec((DIM, DIM), lambda i: (0, 0)),
            pl.BlockSpec((2, DIM), lambda i: (0, 0)),
        ],
        out_specs=[
            pl.BlockSpec((BLK, DIM), lambda i: (i, 0)),
            pl.BlockSpec((2, BLK // 128, 128), lambda i: (0, i, 0)),
            pl.BlockSpec((8, 128), lambda i: (0, 0)),
            pl.BlockSpec((8, 128), lambda i: (0, 0)),
        ],
        out_shape=[
            jax.ShapeDtypeStruct((N_PAD, DIM), jnp.float32),
            jax.ShapeDtypeStruct((2, N_PAD // 128, 128), jnp.float32),
            jax.ShapeDtypeStruct((8, 128), jnp.float32),
            jax.ShapeDtypeStruct((8, 128), jnp.float32),
        ],
    )(x_pad, W, ab)


def _sc_aggregate(h, hsd_flat, ms, md, src, dst):
    mesh = plsc.VectorSubcoreMesh(core_axis_name="c", subcore_axis_name="s")

    @functools.partial(
        pl.kernel,
        mesh=mesh,
        out_type=jax.ShapeDtypeStruct((2, N_NODES, ACC_W), jnp.float32),
        scratch_types=[
            pltpu.VMEM((N_PAD,), jnp.float32),
            pltpu.VMEM((N_PAD,), jnp.float32),
            pltpu.VMEM((16,), jnp.float32),
            pltpu.VMEM((16,), jnp.float32),
            pltpu.VMEM((C,), jnp.int32),
            pltpu.VMEM((C,), jnp.int32),
            pltpu.VMEM((C,), jnp.float32),
            pltpu.VMEM((C, DIM), jnp.float32),
            pltpu.VMEM((C, ACC_W), jnp.float32),
            pltpu.VMEM((ZROWS, ACC_W), jnp.float32),
            pltpu.VMEM_SHARED((N_NODES, ACC_W), jnp.float32),
            pltpu.SemaphoreType.DMA,
        ],
    )
    def body(h_hbm, hsd_hbm, ms_hbm, md_hbm, src_hbm, dst_hbm, out_hbm,
             hs_v, hd_v, m_v, t_v, src_v, dst_v, p_v, rows_v, msg_v,
             zbuf, acc_sh, sem):
        cid = lax.axis_index("c")
        sid = lax.axis_index("s")

        @pl.loop(0, ZROWS)
        def _zrow(r):
            for col in range(ACC_W // 16):
                zbuf[r, pl.ds(col * 16, 16)] = jnp.zeros((16,), jnp.float32)

        row0 = sid * ROWS_PER_SUB

        @pl.loop(0, ROWS_PER_SUB // ZROWS)
        def _zacc(r):
            pltpu.sync_copy(zbuf, acc_sh.at[pl.ds(row0 + r * ZROWS, ZROWS)])

        pltpu.sync_copy(hsd_hbm.at[0], hs_v)
        pltpu.sync_copy(hsd_hbm.at[1], hd_v)
        pltpu.sync_copy(ms_hbm.at[0, pl.ds(0, 16)], m_v)
        pltpu.sync_copy(md_hbm.at[0, pl.ds(0, 16)], t_v)
        msum = m_v[...] + t_v[...]
        m_v[...] = jnp.where(msum >= 0.0, msum, msum * 0.2)

        plsc.subcore_barrier()

        base = (cid * 16 + sid) * EPW

        @pl.loop(0, NCHUNK)
        def _chunk(k):
            off = base + k * C
            pltpu.sync_copy(src_hbm.at[pl.ds(off, C)], src_v)
            pltpu.sync_copy(dst_hbm.at[pl.ds(off, C)], dst_v)
            pltpu.async_copy(h_hbm.at[src_v], rows_v, sem).wait()
            for g in range(C // 16):
                s16 = src_v[pl.ds(g * 16, 16)]
                d16 = dst_v[pl.ds(g * 16, 16)]
                v = plsc.load_gather(hs_v, [s16]) + plsc.load_gather(hd_v, [d16])
                e = jnp.where(v >= 0.0, v, v * 0.2)
                p_v[pl.ds(g * 16, 16)] = jnp.exp(e - m_v[...])

            @pl.loop(0, C)
            def _scale(i):
                iv = jnp.zeros((16,), jnp.int32) + i
                pb = plsc.load_gather(p_v, [iv])
                for col in range(DIM // 16):
                    msg_v[i, pl.ds(col * 16, 16)] = (
                        rows_v[i, pl.ds(col * 16, 16)] * pb)
                msg_v[i, pl.ds(DIM, 16)] = pb

            pltpu.sync_copy(msg_v, acc_sh.at[dst_v], add=True)

        plsc.subcore_barrier()

        @pl.loop(0, ROWS_PER_SUB // ZROWS)
        def _wb(r):
            roff = row0 + r * ZROWS
            pltpu.sync_copy(acc_sh.at[pl.ds(roff, ZROWS)],
                            out_hbm.at[cid, pl.ds(roff, ZROWS)])

    return body(h, hsd_flat, ms, md, src, dst)


def _finish_body(acc_ref, out_ref):
    a = acc_ref[0]
    b = acc_ref[1]
    num = a[:, :DIM] + b[:, :DIM]
    den = a[:, DIM:DIM + 1] + b[:, DIM:DIM + 1]
    out_ref[...] = num / jnp.maximum(den, 1e-16)


def _finish(acc):
    blk = 2000
    return pl.pallas_call(
        _finish_body,
        grid=(N_NODES // blk,),
        in_specs=[pl.BlockSpec((2, blk, ACC_W), lambda i: (0, i, 0))],
        out_specs=pl.BlockSpec((blk, DIM), lambda i: (i, 0)),
        out_shape=jax.ShapeDtypeStruct((N_NODES, DIM), jnp.float32),
    )(acc)


def kernel(x, edge_index, W, a_src, a_dst):
    src = edge_index[0].astype(jnp.int32)
    dst = edge_index[1].astype(jnp.int32)
    x_pad = jnp.pad(x, ((0, N_PAD - N_NODES), (0, 0)))
    ab = jnp.stack([a_src, a_dst])
    h, hsd, ms, md = _prep(x_pad, W, ab)
    hsd_flat = hsd.reshape(2, N_PAD)
    acc = _sc_aggregate(h, hsd_flat, ms, md, src, dst)
    return _finish(acc)


# speedup vs baseline: 23.5763x; 23.5763x over previous
import dataclasses
import functools

import jax
import jax.numpy as jnp
from jax import lax
from jax.experimental import pallas as pl
from jax.experimental.pallas import tpu as pltpu
from jax.experimental.pallas import tpu_sc as plsc

N_NODES = 10000
N_EDGES = 320000
DIM = 128

N_PAD = 10240
BLK = 2048
NROW = N_PAD // 128

NW = 32
EPW = N_EDGES // NW
C = 80
NCHUNK = EPW // C
ROWS_PER_SUB = N_PAD // 16
ZROWS = 80


def _prep_body(x_ref, w_ref, ab_ref, h_ref, hsd_ref, ms_ref, md_ref):
    i = pl.program_id(0)
    h = jnp.dot(x_ref[...], w_ref[...], preferred_element_type=jnp.float32)
    h_ref[...] = h
    hs = jnp.sum(h * ab_ref[0, :][None, :], axis=1)
    hd = jnp.sum(h * ab_ref[1, :][None, :], axis=1)
    hsd_ref[0] = hs.reshape(BLK // 128, 128)
    hsd_ref[1] = hd.reshape(BLK // 128, 128)

    @pl.when(i == 0)
    def _():
        ms_ref[...] = jnp.full((8, 128), -1e30, jnp.float32)
        md_ref[...] = jnp.full((8, 128), -1e30, jnp.float32)

    ms_ref[...] = jnp.maximum(ms_ref[...], jnp.max(hs))
    md_ref[...] = jnp.maximum(md_ref[...], jnp.max(hd))


def _prep(x_pad, W, ab):
    return pl.pallas_call(
        _prep_body,
        grid=(N_PAD // BLK,),
        in_specs=[
            pl.BlockSpec((BLK, DIM), lambda i: (i, 0)),
            pl.BlockSpec((DIM, DIM), lambda i: (0, 0)),
            pl.BlockSpec((2, DIM), lambda i: (0, 0)),
        ],
        out_specs=[
            pl.BlockSpec((BLK, DIM), lambda i: (i, 0)),
            pl.BlockSpec((2, BLK // 128, 128), lambda i: (0, i, 0)),
            pl.BlockSpec((8, 128), lambda i: (0, 0)),
            pl.BlockSpec((8, 128), lambda i: (0, 0)),
        ],
        out_shape=[
            jax.ShapeDtypeStruct((N_PAD, DIM), jnp.float32),
            jax.ShapeDtypeStruct((2, N_PAD // 128, 128), jnp.float32),
            jax.ShapeDtypeStruct((8, 128), jnp.float32),
            jax.ShapeDtypeStruct((8, 128), jnp.float32),
        ],
    )(x_pad, W, ab)


def _sc_aggregate(h, hsd_flat, ms, md, src, dst):
    mesh = plsc.VectorSubcoreMesh(core_axis_name="c", subcore_axis_name="s")
    cp = pltpu.CompilerParams()
    if "needs_layout_passes" in pltpu.CompilerParams.__dataclass_fields__:
        cp = dataclasses.replace(cp, needs_layout_passes=False)

    @functools.partial(
        pl.kernel,
        mesh=mesh,
        compiler_params=cp,
        out_type=[
            jax.ShapeDtypeStruct((2, N_PAD, DIM), jnp.float32),
            jax.ShapeDtypeStruct((NW, NROW, 128), jnp.float32),
        ],
        scratch_types=[
            pltpu.VMEM((N_PAD,), jnp.float32),
            pltpu.VMEM((N_PAD,), jnp.float32),
            pltpu.VMEM((16,), jnp.float32),
            pltpu.VMEM((16,), jnp.float32),
            pltpu.VMEM((C,), jnp.int32),
            pltpu.VMEM((C,), jnp.int32),
            pltpu.VMEM((C,), jnp.float32),
            pltpu.VMEM((C, DIM), jnp.float32),
            pltpu.VMEM((NROW, 128), jnp.float32),
            pltpu.VMEM_SHARED((N_PAD, DIM), jnp.float32),
            pltpu.SemaphoreType.DMA,
        ],
    )
    def body(h_hbm, hsd_hbm, ms_hbm, md_hbm, src_hbm, dst_hbm,
             num_hbm, den_hbm,
             hs_v, hd_v, m_v, t_v, src_v, dst_v, p_v, rows_v, den_v,
             acc_sh, sem):
        cid = lax.axis_index("c")
        sid = lax.axis_index("s")

        @pl.loop(0, C)
        def _zrow(r):
            for col in range(DIM // 16):
                rows_v[r, pl.ds(col * 16, 16)] = jnp.zeros((16,), jnp.float32)

        @pl.loop(0, NROW)
        def _zden(r):
            for col in range(128 // 16):
                den_v[r, pl.ds(col * 16, 16)] = jnp.zeros((16,), jnp.float32)

        row0 = sid * ROWS_PER_SUB

        @pl.loop(0, ROWS_PER_SUB // ZROWS)
        def _zacc(r):
            pltpu.sync_copy(rows_v, acc_sh.at[pl.ds(row0 + r * ZROWS, ZROWS)])

        pltpu.sync_copy(hsd_hbm.at[0], hs_v)
        pltpu.sync_copy(hsd_hbm.at[1], hd_v)
        pltpu.sync_copy(ms_hbm.at[0, pl.ds(0, 16)], m_v)
        pltpu.sync_copy(md_hbm.at[0, pl.ds(0, 16)], t_v)
        msum = m_v[...] + t_v[...]
        m_v[...] = jnp.where(msum >= 0.0, msum, msum * 0.2)

        plsc.subcore_barrier()

        base = (cid * 16 + sid) * EPW

        @pl.loop(0, NCHUNK)
        def _chunk(k):
            off = base + k * C
            pltpu.sync_copy(src_hbm.at[pl.ds(off, C)], src_v)
            pltpu.sync_copy(dst_hbm.at[pl.ds(off, C)], dst_v)
            pltpu.async_copy(h_hbm.at[src_v], rows_v, sem).wait()
            for g in range(C // 16):
                s16 = src_v[pl.ds(g * 16, 16)]
                d16 = dst_v[pl.ds(g * 16, 16)]
                v = plsc.load_gather(hs_v, [s16]) + plsc.load_gather(hd_v, [d16])
                e = jnp.where(v >= 0.0, v, v * 0.2)
                p16 = jnp.exp(e - m_v[...])
                p_v[pl.ds(g * 16, 16)] = p16
                plsc.addupdate_scatter(
                    den_v,
                    [lax.shift_right_logical(d16, 7), lax.bitwise_and(d16, 127)],
                    p16)

            @pl.loop(0, C)
            def _scale(i):
                iv = jnp.zeros((16,), jnp.int32) + i
                pb = plsc.load_gather(p_v, [iv])
                for col in range(DIM // 16):
                    rows_v[i, pl.ds(col * 16, 16)] = (
                        rows_v[i, pl.ds(col * 16, 16)] * pb)

            pltpu.sync_copy(rows_v, acc_sh.at[dst_v], add=True)

        wid = cid * 16 + sid
        pltpu.sync_copy(den_v, den_hbm.at[wid])

        plsc.subcore_barrier()

        @pl.loop(0, ROWS_PER_SUB // ZROWS)
        def _wb(r):
            roff = row0 + r * ZROWS
            pltpu.sync_copy(acc_sh.at[pl.ds(roff, ZROWS)],
                            num_hbm.at[cid, pl.ds(roff, ZROWS)])

    return body(h, hsd_flat, ms, md, src, dst)


def _den_reduce_body(denp_ref, den_ref):
    den_ref[...] = jnp.sum(denp_ref[...], axis=0)


def _den_reduce(den_parts):
    return pl.pallas_call(
        _den_reduce_body,
        grid=(NROW // 16,),
        in_specs=[pl.BlockSpec((NW, 16, 128), lambda i: (0, i, 0))],
        out_specs=pl.BlockSpec((16, 128), lambda i: (i, 0)),
        out_shape=jax.ShapeDtypeStruct((NROW, 128), jnp.float32),
    )(den_parts)


def _finish_body(num_ref, den_ref, out_ref):
    s = num_ref[0] + num_ref[1]
    den = jnp.maximum(den_ref[...], 1e-16)
    out_ref[...] = s / den


def _finish(num, den_col):
    return pl.pallas_call(
        _finish_body,
        grid=(N_PAD // BLK,),
        in_specs=[
            pl.BlockSpec((2, BLK, DIM), lambda i: (0, i, 0)),
            pl.BlockSpec((BLK, 1), lambda i: (i, 0)),
        ],
        out_specs=pl.BlockSpec((BLK, DIM), lambda i: (i, 0)),
        out_shape=jax.ShapeDtypeStruct((N_PAD, DIM), jnp.float32),
    )(num, den_col)


def kernel(x, edge_index, W, a_src, a_dst):
    src = edge_index[0].astype(jnp.int32)
    dst = edge_index[1].astype(jnp.int32)
    x_pad = jnp.pad(x, ((0, N_PAD - N_NODES), (0, 0)))
    ab = jnp.stack([a_src, a_dst])
    h, hsd, ms, md = _prep(x_pad, W, ab)
    hsd_flat = hsd.reshape(2, N_PAD)
    num, den_parts = _sc_aggregate(h, hsd_flat, ms, md, src, dst)
    den = _den_reduce(den_parts)
    out = _finish(num, den.reshape(N_PAD, 1))
    return out[:N_NODES]
